# Initial kernel scaffold; baseline (speedup 1.0000x reference)
import jax
import jax.numpy as jnp
from jax.experimental import pallas as pl


def kernel(x, W1, D1, W2, D2, W3, D3, P1, P2, P3, Wh):
    raise NotImplementedError("write your pallas kernel here")


# speedup vs baseline: 1.5764x; 1.5764x over previous
import jax
import jax.numpy as jnp
from jax.experimental import pallas as pl
from jax.experimental.pallas import tpu as pltpu

EPS = 1e-6
NEG = 0.2
K = 16


def _vnn_kernel(xtt_ref, xtf_ref, xff_ref, w1_ref, d1_ref, p1_ref, p3_ref,
                s_ref, sb_ref, wh_ref, out_ref, bval, bp):
    t = pl.program_id(1)
    num_t = pl.num_programs(1)
    ctr = xtt_ref[0]
    xtf = xtf_ref[0]
    xff = xff_ref[0]
    R = ctr.shape[0]
    N = xtf.shape[0]

    @pl.when(t == 0)
    def _init():
        bval[...] = jnp.full(bval.shape, -jnp.inf, jnp.float32)
        bp[...] = jnp.zeros(bp.shape, jnp.float32)

    g = jnp.dot(ctr.astype(jnp.bfloat16), xff.astype(jnp.bfloat16),
                preferred_element_type=jnp.float32)
    xx_r = jnp.sum(ctr * ctr, axis=1, keepdims=True)
    xx_c = jnp.sum(xff * xff, axis=0, keepdims=True)
    dist = 2.0 * g - xx_r - xx_c

    w1 = w1_ref[...]
    d1w = d1_ref[...]
    p1 = p1_ref[...]
    s = s_ref[...]
    sb = sb_ref[...]

    ci = jax.lax.broadcasted_iota(jnp.int32, (R, N), 1)
    best_dot = jnp.full((R, 32), -jnp.inf, jnp.float32)
    best_h = jnp.zeros((R, 96), jnp.float32)

    for _ in range(K):
        m = jnp.max(dist, axis=1, keepdims=True)
        nidx = jnp.min(jnp.where(dist == m, ci, N), axis=1, keepdims=True)
        oh = ci == nidx
        ohf = oh.astype(jnp.float32)
        nbr = jnp.dot(ohf, xtf, preferred_element_type=jnp.float32, precision=jax.lax.Precision.HIGHEST)
        dist = jnp.where(oh, -jnp.inf, dist)

        f = jnp.concatenate([nbr - ctr, ctr], axis=1)
        fb = f.astype(jnp.bfloat16)
        p = jnp.dot(fb, w1, preferred_element_type=jnp.float32)
        d = jnp.dot(fb, d1w, preferred_element_type=jnp.float32)
        dotc = jnp.dot(p * d, s, preferred_element_type=jnp.float32, precision=jax.lax.Precision.HIGHEST)
        dns = jnp.dot(d * d, s, preferred_element_type=jnp.float32, precision=jax.lax.Precision.HIGHEST)
        dot_b = jnp.dot(dotc, sb, preferred_element_type=jnp.float32, precision=jax.lax.Precision.HIGHEST)
        dns_b = jnp.dot(dns, sb, preferred_element_type=jnp.float32, precision=jax.lax.Precision.HIGHEST)
        hneg = p - (dot_b / (dns_b + EPS)) * d
        hp = NEG * p + (1.0 - NEG) * jnp.where(dot_b >= 0, p, hneg)

        q = jnp.dot(hp.astype(jnp.bfloat16), p1,
                    preferred_element_type=jnp.float32)
        dotp = jnp.dot(hp * q, s, preferred_element_type=jnp.float32, precision=jax.lax.Precision.HIGHEST)
        upd = dotp > best_dot
        best_dot = jnp.where(upd, dotp, best_dot)
        updf = jnp.dot(upd.astype(jnp.float32), sb,
                       preferred_element_type=jnp.float32, precision=jax.lax.Precision.HIGHEST)
        best_h = updf * hp + (1.0 - updf) * best_h

    r = jnp.dot(best_h.astype(jnp.bfloat16), p3_ref[...],
                preferred_element_type=jnp.float32)
    dotn = jnp.dot(best_h * r, s, preferred_element_type=jnp.float32, precision=jax.lax.Precision.HIGHEST)
    tmx = jnp.max(dotn, axis=0, keepdims=True)
    ri = jax.lax.broadcasted_iota(jnp.int32, (R, 32), 0)
    rmin = jnp.min(jnp.where(dotn == tmx, ri, R), axis=0, keepdims=True)
    ohn = (ri == rmin).astype(jnp.float32)
    oh96 = jnp.dot(ohn, sb, preferred_element_type=jnp.float32, precision=jax.lax.Precision.HIGHEST)
    pcand = jnp.sum(oh96 * best_h, axis=0, keepdims=True)

    cur_val = bval[0:1, 0:32]
    cur_p = bp[0:1, 0:96]
    upd2 = tmx > cur_val
    bval[0:1, 0:32] = jnp.where(upd2, tmx, cur_val)
    u96 = jnp.dot(upd2.astype(jnp.float32), sb,
                  preferred_element_type=jnp.float32, precision=jax.lax.Precision.HIGHEST)
    bp[0:1, 0:96] = u96 * pcand + (1.0 - u96) * cur_p

    @pl.when(t == num_t - 1)
    def _finish():
        res = jnp.dot(bp[0:1, 0:96].astype(jnp.bfloat16), wh_ref[...],
                      preferred_element_type=jnp.float32)
        out_ref[0] = res


def kernel(x, W1, D1, W2, D2, W3, D3, P1, P2, P3, Wh):
    B, C, three, N = x.shape
    R = 256
    T = N // R
    f32 = jnp.float32

    xf = x.reshape(B, C * three, N)
    xt = jnp.transpose(xf, (0, 2, 1))

    eye3 = jnp.eye(3, dtype=f32)
    def expand(w):
        o, c = w.shape
        return jnp.einsum('oc,de->cdoe', w, eye3).reshape(3 * c, 3 * o)
    bf = jnp.bfloat16
    w1e = expand(W1).astype(bf)
    d1e = expand(D1).astype(bf)
    p1e = expand(P1).astype(bf)
    p3e = expand(P3).astype(bf)
    whe = expand(Wh).astype(bf)
    s = jnp.repeat(jnp.eye(32, dtype=f32), 3, axis=0)
    sb = s.T

    full = lambda shape: pl.BlockSpec(shape, lambda b, t: (0,) * len(shape))
    out = pl.pallas_call(
        _vnn_kernel,
        grid=(B, T),
        in_specs=[
            pl.BlockSpec((1, R, 9), lambda b, t: (b, t, 0)),
            pl.BlockSpec((1, N, 9), lambda b, t: (b, 0, 0)),
            pl.BlockSpec((1, 9, N), lambda b, t: (b, 0, 0)),
            full((18, 96)), full((18, 96)), full((96, 96)), full((96, 96)),
            full((96, 32)), full((32, 96)), full((96, 9)),
        ],
        out_specs=pl.BlockSpec((1, 1, 9), lambda b, t: (b, 0, 0)),
        out_shape=jax.ShapeDtypeStruct((B, 1, 9), f32),
        scratch_shapes=[pltpu.VMEM((8, 128), f32), pltpu.VMEM((8, 128), f32)],
        compiler_params=pltpu.CompilerParams(
            dimension_semantics=("arbitrary", "arbitrary")),
    )(xt, xt, xf, w1e, d1e, p1e, p3e, s, sb, whe)
    return out.reshape(B, 3, 3)


# speedup vs baseline: 4.8407x; 3.0708x over previous
import functools

import jax
import jax.numpy as jnp
from jax.experimental import pallas as pl
from jax.experimental.pallas import tpu as pltpu

EPS = 1e-6
NEG = 0.2
K = 16
BF = jnp.bfloat16
F32 = jnp.float32


def _split3(v):
    p1 = v.astype(BF)
    r1 = v - p1.astype(F32)
    p2 = r1.astype(BF)
    p3 = (r1 - p2.astype(F32)).astype(BF)
    return p1, p2, p3


def _sum3(m, s3):
    return jnp.dot(m, s3, preferred_element_type=F32,
                   precision=jax.lax.Precision.HIGHEST)


def _vnn_kernel(xtn_ref, xff_ref, wd_ref, p1_ref, p3_ref,
                s3_ref, sb3_ref, sb1_ref, wh_ref, out_ref, bval, bp, *, R):
    t = pl.program_id(1)
    num_t = pl.num_programs(1)
    xtn = xtn_ref[0]
    xff = xff_ref[0]
    N = xtn.shape[0]
    ctr = xtn_ref[0, pl.ds(t * R, R), :]
    t1, t2, t3 = _split3(xtn)
    tab = jnp.concatenate([t1, t2, t3], axis=1)

    @pl.when(t == 0)
    def _init():
        bval[...] = jnp.full(bval.shape, -jnp.inf, F32)
        bp[...] = jnp.zeros(bp.shape, F32)

    g = jnp.dot(ctr.astype(BF), xff.astype(BF),
                preferred_element_type=F32)
    xx_r = jnp.sum(ctr * ctr, axis=1, keepdims=True)
    xx_c = jnp.sum(xff * xff, axis=0, keepdims=True)
    dist = (2.0 * g - xx_c) - xx_r

    wd = wd_ref[...]
    p1 = p1_ref[...]
    s3 = s3_ref[...]
    sb3 = sb3_ref[...]
    sb1 = sb1_ref[...]
    ci = jax.lax.broadcasted_iota(jnp.int32, (R, N), 1)
    best_dot = jnp.full((R, 32), -jnp.inf, F32)
    best_h = jnp.zeros((R, 96), F32)

    def bcast96(v32):
        a, b, c = _split3(v32)
        return jnp.dot(jnp.concatenate([a, b, c], axis=1), sb3,
                       preferred_element_type=F32)

    for _it in range(K):
        m = jnp.max(dist, axis=1, keepdims=True)
        nidx = jnp.min(jnp.where(dist == m, ci, N), axis=1, keepdims=True)
        oh = ci == nidx
        nb3 = jnp.dot(oh.astype(BF), tab, preferred_element_type=F32)
        nbr = (nb3[:, 0:9] + nb3[:, 9:18]) + nb3[:, 18:27]
        dist = jnp.where(oh, -jnp.inf, dist)

        f = jnp.concatenate([nbr - ctr, ctr], axis=1)
        pd = jnp.dot(f.astype(BF), wd, preferred_element_type=F32)
        p = pd[:, 0:96]
        d = pd[:, 96:192]
        dotc = _sum3(p * d, s3)
        dns = _sum3(d * d, s3)
        dot_b = bcast96(dotc)
        dns_b = bcast96(dns)
        hneg = p - (dot_b / (dns_b + EPS)) * d
        hp = NEG * p + (1.0 - NEG) * jnp.where(dot_b >= 0, p, hneg)

        q = jnp.dot(hp.astype(BF), p1, preferred_element_type=F32)
        dotp = _sum3(hp * q, s3)
        upd = dotp > best_dot
        best_dot = jnp.where(upd, dotp, best_dot)
        updf = jnp.dot(upd.astype(BF), sb1, preferred_element_type=F32)
        best_h = updf * hp + (1.0 - updf) * best_h

    r = jnp.dot(best_h.astype(BF), p3_ref[...], preferred_element_type=F32)
    dotn = _sum3(best_h * r, s3)
    tmx = jnp.max(dotn, axis=0, keepdims=True)
    ri = jax.lax.broadcasted_iota(jnp.int32, (R, 32), 0)
    rmin = jnp.min(jnp.where(dotn == tmx, ri, R), axis=0, keepdims=True)
    ohn = (ri == rmin).astype(BF)
    oh96 = jnp.dot(ohn, sb1, preferred_element_type=F32)
    pcand = jnp.sum(oh96 * best_h, axis=0, keepdims=True)

    cur_val = bval[0:1, 0:32]
    cur_p = bp[0:1, 0:96]
    upd2 = tmx > cur_val
    bval[0:1, 0:32] = jnp.where(upd2, tmx, cur_val)
    u96 = jnp.dot(upd2.astype(BF), sb1, preferred_element_type=F32)
    bp[0:1, 0:96] = u96 * pcand + (1.0 - u96) * cur_p

    @pl.when(t == num_t - 1)
    def _finish():
        res = jnp.dot(bp[0:1, 0:96].astype(BF), wh_ref[...],
                      preferred_element_type=F32)
        out_ref[0] = res


def kernel(x, W1, D1, W2, D2, W3, D3, P1, P2, P3, Wh):
    B, C, three, N = x.shape
    R = 256
    T = N // R

    xf = x.reshape(B, C * three, N)
    xt = jnp.transpose(xf, (0, 2, 1))

    eye3 = jnp.eye(3, dtype=F32)
    def expand(w):
        o, c = w.shape
        return jnp.einsum('oc,de->cdoe', w, eye3).reshape(3 * c, 3 * o)
    wd = jnp.concatenate([expand(W1), expand(D1)], axis=1).astype(BF)
    p1e = expand(P1).astype(BF)
    p3e = expand(P3).astype(BF)
    whe = expand(Wh).astype(BF)
    s = jnp.repeat(jnp.eye(32, dtype=F32), 3, axis=0)
    s3 = s
    sb1 = s.T.astype(BF)
    sb3 = jnp.concatenate([s.T, s.T, s.T], axis=0).astype(BF)

    full = lambda shape: pl.BlockSpec(shape, lambda b, t: (0,) * len(shape))
    out = pl.pallas_call(
        functools.partial(_vnn_kernel, R=R),
        grid=(B, T),
        in_specs=[
            pl.BlockSpec((1, N, 9), lambda b, t: (b, 0, 0)),
            pl.BlockSpec((1, 9, N), lambda b, t: (b, 0, 0)),
            full((18, 192)), full((96, 96)), full((96, 96)),
            full((96, 32)), full((96, 96)), full((32, 96)), full((96, 9)),
        ],
        out_specs=pl.BlockSpec((1, 1, 9), lambda b, t: (b, 0, 0)),
        out_shape=jax.ShapeDtypeStruct((B, 1, 9), F32),
        scratch_shapes=[pltpu.VMEM((8, 128), F32), pltpu.VMEM((8, 128), F32)],
        compiler_params=pltpu.CompilerParams(
            dimension_semantics=("arbitrary", "arbitrary")),
    )(xt, xf, wd, p1e, p3e, s3, sb3, sb1, whe)
    return out.reshape(B, 3, 3)
